# baseline (device time: 39253 ns/iter reference)
import jax
import jax.numpy as jnp
from jax import lax
from jax.experimental import pallas as pl
from jax.experimental.pallas import tpu as pltpu


def kernel(Q, K, V):
    b, q, h, d = Q.shape
    kv = K.shape[1]
    scale = d ** -0.5

    def body(q_ref, k_ref, v_ref, out_ref, o_comm, ml_comm, o_sems, ml_sems):
        my_x = lax.axis_index("x")
        my_y = lax.axis_index("y")
        nbr = (my_x, 1 - my_y)

        barrier = pltpu.get_barrier_semaphore()
        pl.semaphore_signal(
            barrier, inc=1, device_id=nbr, device_id_type=pl.DeviceIdType.MESH
        )
        pl.semaphore_wait(barrier, 1)

        s = jnp.sum(q_ref[...] * k_ref[...], axis=-1) * scale
        m = jnp.max(s, axis=1, keepdims=True)
        p = jnp.exp(s - m)
        l = jnp.sum(p, axis=1, keepdims=True)
        o = jnp.sum(p[..., None] * v_ref[...], axis=1)

        m_a = m[:, 0, :]
        l_a = l[:, 0, :]

        o_comm[0] = o
        ml_comm[0, 0] = m_a
        ml_comm[0, 1] = l_a

        o_rdma = pltpu.make_async_remote_copy(
            src_ref=o_comm.at[0],
            dst_ref=o_comm.at[1],
            send_sem=o_sems.at[0],
            recv_sem=o_sems.at[1],
            device_id=nbr,
            device_id_type=pl.DeviceIdType.MESH,
        )
        ml_rdma = pltpu.make_async_remote_copy(
            src_ref=ml_comm.at[0],
            dst_ref=ml_comm.at[1],
            send_sem=ml_sems.at[0],
            recv_sem=ml_sems.at[1],
            device_id=nbr,
            device_id_type=pl.DeviceIdType.MESH,
        )
        o_rdma.start()
        ml_rdma.start()
        o_rdma.wait()
        ml_rdma.wait()

        m_b = ml_comm[1, 0]
        l_b = ml_comm[1, 1]
        o_b = o_comm[1]
        m_g = jnp.maximum(m_a, m_b)
        ea = jnp.exp(m_a - m_g)
        eb = jnp.exp(m_b - m_g)
        denom = l_a * ea + l_b * eb
        o_full = (o * ea[..., None] + o_b * eb[..., None]) / denom[..., None]
        out_ref[...] = o_full[:, None, :, :]

    return pl.pallas_call(
        body,
        out_shape=jax.ShapeDtypeStruct((b, q, h, d), jnp.float32),
        in_specs=[
            pl.BlockSpec(memory_space=pltpu.VMEM),
            pl.BlockSpec(memory_space=pltpu.VMEM),
            pl.BlockSpec(memory_space=pltpu.VMEM),
        ],
        out_specs=pl.BlockSpec(memory_space=pltpu.VMEM),
        scratch_shapes=[
            pltpu.VMEM((2, b, h, d), jnp.float32),
            pltpu.VMEM((2, 2, b, h), jnp.float32),
            pltpu.SemaphoreType.DMA((2,)),
            pltpu.SemaphoreType.DMA((2,)),
        ],
        compiler_params=pltpu.CompilerParams(collective_id=0),
    )(Q, K, V)


# device time: 38869 ns/iter; 1.0099x vs baseline; 1.0099x over previous
import jax
import jax.numpy as jnp
from jax import lax
from jax.experimental import pallas as pl
from jax.experimental.pallas import tpu as pltpu


def kernel(Q, K, V):
    b, q, h, d = Q.shape
    kv = K.shape[1]
    scale = d ** -0.5

    def body(q_ref, k_ref, v_ref, out_ref, o_comm, l_comm, o_sems, l_sems):
        my_x = lax.axis_index("x")
        my_y = lax.axis_index("y")
        nbr = (my_x, 1 - my_y)

        barrier = pltpu.get_barrier_semaphore()
        pl.semaphore_signal(
            barrier, inc=1, device_id=nbr, device_id_type=pl.DeviceIdType.MESH
        )
        pl.semaphore_wait(barrier, 1)

        ii = lax.broadcasted_iota(jnp.int32, (h, h), 0)
        jj = lax.broadcasted_iota(jnp.int32, (h, h), 1)
        eyem = (ii == jj).astype(jnp.float32)

        o_list = []
        l_list = []
        for bi in range(b):
            kb = k_ref[bi].reshape(kv * h, d)
            qb = q_ref[bi, 0]
            g = lax.dot_general(
                kb, qb,
                dimension_numbers=(((1,), (1,)), ((), ())),
                preferred_element_type=jnp.float32,
            )
            gr = g.reshape(kv, h, h)
            s_b = jnp.sum(gr * eyem, axis=-1) * scale
            p_b = jnp.exp(s_b)
            l_b = jnp.sum(p_b, axis=0)
            w = (p_b[:, :, None] * eyem).reshape(kv * h, h)
            vb = v_ref[bi].reshape(kv * h, d)
            o_b = lax.dot_general(
                w, vb,
                dimension_numbers=(((0,), (0,)), ((), ())),
                preferred_element_type=jnp.float32,
            )
            o_list.append(o_b)
            l_list.append(l_b)
        o = jnp.stack(o_list)
        l = jnp.stack(l_list)

        o_comm[0] = o
        l_comm[0] = l

        o_rdma = pltpu.make_async_remote_copy(
            src_ref=o_comm.at[0],
            dst_ref=o_comm.at[1],
            send_sem=o_sems.at[0],
            recv_sem=o_sems.at[1],
            device_id=nbr,
            device_id_type=pl.DeviceIdType.MESH,
        )
        l_rdma = pltpu.make_async_remote_copy(
            src_ref=l_comm.at[0],
            dst_ref=l_comm.at[1],
            send_sem=l_sems.at[0],
            recv_sem=l_sems.at[1],
            device_id=nbr,
            device_id_type=pl.DeviceIdType.MESH,
        )
        o_rdma.start()
        l_rdma.start()
        o_rdma.wait()
        l_rdma.wait()

        denom = l + l_comm[1]
        o_full = (o + o_comm[1]) / denom[..., None]
        out_ref[...] = o_full[:, None, :, :]

    return pl.pallas_call(
        body,
        out_shape=jax.ShapeDtypeStruct((b, q, h, d), jnp.float32),
        in_specs=[
            pl.BlockSpec(memory_space=pltpu.VMEM),
            pl.BlockSpec(memory_space=pltpu.VMEM),
            pl.BlockSpec(memory_space=pltpu.VMEM),
        ],
        out_specs=pl.BlockSpec(memory_space=pltpu.VMEM),
        scratch_shapes=[
            pltpu.VMEM((2, b, h, d), jnp.float32),
            pltpu.VMEM((2, b, h), jnp.float32),
            pltpu.SemaphoreType.DMA((2,)),
            pltpu.SemaphoreType.DMA((2,)),
        ],
        compiler_params=pltpu.CompilerParams(collective_id=0),
    )(Q, K, V)


# device time: 18831 ns/iter; 2.0845x vs baseline; 2.0641x over previous
import jax
import jax.numpy as jnp
from jax import lax
from jax.experimental import pallas as pl
from jax.experimental.pallas import tpu as pltpu


def kernel(Q, K, V):
    b, q, h, d = Q.shape
    kv = K.shape[1]
    scale = d ** -0.5

    Kt = jnp.transpose(K, (0, 2, 3, 1))
    Vt = jnp.transpose(V, (0, 2, 3, 1))
    eye8 = jnp.eye(h, dtype=jnp.float32)
    Qbd = (Q[:, 0, :, None, :] * (eye8 * scale)[None, :, :, None]).reshape(
        b, h, h * d
    )

    def body(qbd_ref, kt_ref, vt_ref, out_ref, o_comm, l_comm, o_sems, l_sems):
        my_x = lax.axis_index("x")
        my_y = lax.axis_index("y")
        nbr = (my_x, 1 - my_y)

        barrier = pltpu.get_barrier_semaphore()
        pl.semaphore_signal(
            barrier, inc=1, device_id=nbr, device_id_type=pl.DeviceIdType.MESH
        )
        pl.semaphore_wait(barrier, 1)

        ii = lax.broadcasted_iota(jnp.int32, (h, h), 0)
        jj = lax.broadcasted_iota(jnp.int32, (h, h), 1)
        eyem = (ii == jj).astype(jnp.float32)

        o_list = []
        l_list = []
        for bi in range(b):
            k2 = kt_ref[bi].reshape(h * d, kv)
            s_b = lax.dot_general(
                qbd_ref[bi], k2,
                dimension_numbers=(((1,), (0,)), ((), ())),
                preferred_element_type=jnp.float32,
            )
            p_b = jnp.exp(s_b)
            l_b = jnp.sum(p_b, axis=1)
            v2 = vt_ref[bi].reshape(h * d, kv)
            gv = lax.dot_general(
                v2, p_b,
                dimension_numbers=(((1,), (1,)), ((), ())),
                preferred_element_type=jnp.float32,
            )
            o_b = jnp.sum(
                gv.reshape(h, d, h) * eyem[:, None, :], axis=-1
            )
            o_list.append(o_b)
            l_list.append(l_b)
        o = jnp.stack(o_list)
        l = jnp.stack(l_list)

        o_comm[0] = o
        l_comm[0] = l

        o_rdma = pltpu.make_async_remote_copy(
            src_ref=o_comm.at[0],
            dst_ref=o_comm.at[1],
            send_sem=o_sems.at[0],
            recv_sem=o_sems.at[1],
            device_id=nbr,
            device_id_type=pl.DeviceIdType.MESH,
        )
        l_rdma = pltpu.make_async_remote_copy(
            src_ref=l_comm.at[0],
            dst_ref=l_comm.at[1],
            send_sem=l_sems.at[0],
            recv_sem=l_sems.at[1],
            device_id=nbr,
            device_id_type=pl.DeviceIdType.MESH,
        )
        o_rdma.start()
        l_rdma.start()
        o_rdma.wait()
        l_rdma.wait()

        denom = l + l_comm[1]
        o_full = (o + o_comm[1]) / denom[..., None]
        out_ref[...] = o_full[:, None, :, :]

    return pl.pallas_call(
        body,
        out_shape=jax.ShapeDtypeStruct((b, q, h, d), jnp.float32),
        in_specs=[
            pl.BlockSpec(memory_space=pltpu.VMEM),
            pl.BlockSpec(memory_space=pltpu.VMEM),
            pl.BlockSpec(memory_space=pltpu.VMEM),
        ],
        out_specs=pl.BlockSpec(memory_space=pltpu.VMEM),
        scratch_shapes=[
            pltpu.VMEM((2, b, h, d), jnp.float32),
            pltpu.VMEM((2, b, h), jnp.float32),
            pltpu.SemaphoreType.DMA((2,)),
            pltpu.SemaphoreType.DMA((2,)),
        ],
        compiler_params=pltpu.CompilerParams(collective_id=0),
    )(Qbd, Kt, Vt)
